# baseline (speedup 1.0000x reference)
import functools

import jax
import jax.numpy as jnp
from jax.experimental import pallas as pl
from jax.experimental.pallas import tpu as pltpu

C = 256
CK = C + 128


def _elu1(y):
    return jnp.where(y > 0, y + 1.0, jnp.exp(y))


def _reduce_body(x_ref, wkT_ref, wvT_ref, woT_ref, acc_ref, b_ref, *, steps):
    j = pl.program_id(0)
    xb = x_ref[...].astype(jnp.bfloat16)
    k = _elu1(jnp.dot(xb, wkT_ref[...], preferred_element_type=jnp.float32))
    v = jnp.dot(xb, wvT_ref[...], preferred_element_type=jnp.float32)
    k16 = k.astype(jnp.bfloat16)
    v_ext = jnp.concatenate(
        [v.astype(jnp.bfloat16), jnp.ones((xb.shape[0], 128), jnp.bfloat16)], axis=1
    )
    contrib = jax.lax.dot_general(
        k16, v_ext, (((0,), (0,)), ((), ())), preferred_element_type=jnp.float32
    )

    @pl.when(j == 0)
    def _():
        acc_ref[...] = contrib

    @pl.when(j != 0)
    def _():
        acc_ref[...] += contrib

    @pl.when(j == steps - 1)
    def _():
        kvs = acc_ref[...]
        m = jnp.dot(
            kvs[:, :C].astype(jnp.bfloat16),
            woT_ref[...],
            preferred_element_type=jnp.float32,
        )
        b_ref[...] = jnp.concatenate([m, kvs[:, C:]], axis=1)


def _map_body(x_ref, wqT_ref, b_ref, bo_ref, o_ref):
    xb = x_ref[...].astype(jnp.bfloat16)
    q = _elu1(jnp.dot(xb, wqT_ref[...], preferred_element_type=jnp.float32))
    g = jnp.dot(
        q.astype(jnp.bfloat16),
        b_ref[...].astype(jnp.bfloat16),
        preferred_element_type=jnp.float32,
    )
    z = 1.0 / jnp.maximum(g[:, C:], 1e-6)
    z256 = jnp.concatenate([z, z], axis=1)
    o_ref[...] = g[:, :C] * z256 + bo_ref[...]


@functools.partial(jax.jit, static_argnames=("interpret",))
def kernel(x, Wq, Wk, Wv, Wo, bo, interpret=False):
    n = x.shape[0]
    bn = 8192
    bm = 8192
    s = n // bn
    p = n // bm

    wqT = Wq.T.astype(jnp.bfloat16)
    wkT = Wk.T.astype(jnp.bfloat16)
    wvT = Wv.T.astype(jnp.bfloat16)
    woT = Wo.T.astype(jnp.bfloat16)
    bo2 = bo.reshape(1, C)

    _, bmat = pl.pallas_call(
        functools.partial(_reduce_body, steps=s),
        grid=(s,),
        in_specs=[
            pl.BlockSpec((bn, C), lambda j: (j, 0)),
            pl.BlockSpec((C, C), lambda j: (0, 0)),
            pl.BlockSpec((C, C), lambda j: (0, 0)),
            pl.BlockSpec((C, C), lambda j: (0, 0)),
        ],
        out_specs=[
            pl.BlockSpec((C, CK), lambda j: (0, 0)),
            pl.BlockSpec((C, CK), lambda j: (0, 0)),
        ],
        out_shape=[
            jax.ShapeDtypeStruct((C, CK), jnp.float32),
            jax.ShapeDtypeStruct((C, CK), jnp.float32),
        ],
        compiler_params=pltpu.CompilerParams(
            dimension_semantics=("arbitrary",),
            vmem_limit_bytes=56 * 1024 * 1024,
        ),
        name="lga_reduce",
        interpret=interpret,
    )(x, wkT, wvT, woT)

    out = pl.pallas_call(
        _map_body,
        grid=(p,),
        in_specs=[
            pl.BlockSpec((bm, C), lambda j: (j, 0)),
            pl.BlockSpec((C, C), lambda j: (0, 0)),
            pl.BlockSpec((C, CK), lambda j: (0, 0)),
            pl.BlockSpec((1, C), lambda j: (0, 0)),
        ],
        out_specs=pl.BlockSpec((bm, C), lambda j: (j, 0)),
        out_shape=jax.ShapeDtypeStruct((n, C), jnp.float32),
        compiler_params=pltpu.CompilerParams(
            dimension_semantics=("parallel",),
            vmem_limit_bytes=56 * 1024 * 1024,
        ),
        name="lga_map",
        interpret=interpret,
    )(x, wqT, bmat, bo2)

    return out


# speedup vs baseline: 1.0919x; 1.0919x over previous
import functools

import jax
import jax.numpy as jnp
from jax.experimental import pallas as pl
from jax.experimental.pallas import tpu as pltpu

C = 256
CK = C + 128


def _elu1(y):
    return jnp.where(y > 0, y + 1.0, jnp.exp(y))


def _reduce_body(
    x_ref, wqT_ref, wkT_ref, wvT_ref, woT_ref, q_ref, acc_ref, ks_ref, b_ref, *, steps
):
    j = pl.program_id(0)
    xb = x_ref[...].astype(jnp.bfloat16)
    q = _elu1(jnp.dot(xb, wqT_ref[...], preferred_element_type=jnp.float32))
    q_ref[...] = q.astype(jnp.bfloat16)
    k = _elu1(jnp.dot(xb, wkT_ref[...], preferred_element_type=jnp.float32))
    v = jnp.dot(xb, wvT_ref[...], preferred_element_type=jnp.float32)
    contrib = jax.lax.dot_general(
        k.astype(jnp.bfloat16),
        v.astype(jnp.bfloat16),
        (((0,), (0,)), ((), ())),
        preferred_element_type=jnp.float32,
    )
    ks_part = jnp.sum(k.reshape(-1, 8, C), axis=0)

    @pl.when(j == 0)
    def _():
        acc_ref[...] = contrib
        ks_ref[...] = ks_part

    @pl.when(j != 0)
    def _():
        acc_ref[...] += contrib
        ks_ref[...] += ks_part

    @pl.when(j == steps - 1)
    def _():
        m = jnp.dot(
            acc_ref[...].astype(jnp.bfloat16),
            woT_ref[...],
            preferred_element_type=jnp.float32,
        )
        kcol = jax.lax.dot_general(
            ks_ref[...],
            jnp.ones((8, 128), jnp.float32),
            (((0,), (0,)), ((), ())),
            preferred_element_type=jnp.float32,
        )
        b_ref[...] = jnp.concatenate([m, kcol], axis=1)


def _map_body(q_ref, b_ref, bo_ref, o_ref):
    g = jnp.dot(
        q_ref[...], b_ref[...].astype(jnp.bfloat16), preferred_element_type=jnp.float32
    )
    z = 1.0 / jnp.maximum(g[:, C:], 1e-6)
    z256 = jnp.concatenate([z, z], axis=1)
    o_ref[...] = g[:, :C] * z256 + bo_ref[...]


@functools.partial(jax.jit, static_argnames=("interpret",))
def kernel(x, Wq, Wk, Wv, Wo, bo, interpret=False):
    n = x.shape[0]
    bn = 8192
    bm = 8192
    s = n // bn
    p = n // bm

    wqT = Wq.T.astype(jnp.bfloat16)
    wkT = Wk.T.astype(jnp.bfloat16)
    wvT = Wv.T.astype(jnp.bfloat16)
    woT = Wo.T.astype(jnp.bfloat16)
    bo2 = bo.reshape(1, C)

    q16, _, _, bmat = pl.pallas_call(
        functools.partial(_reduce_body, steps=s),
        grid=(s,),
        in_specs=[
            pl.BlockSpec((bn, C), lambda j: (j, 0)),
            pl.BlockSpec((C, C), lambda j: (0, 0)),
            pl.BlockSpec((C, C), lambda j: (0, 0)),
            pl.BlockSpec((C, C), lambda j: (0, 0)),
            pl.BlockSpec((C, C), lambda j: (0, 0)),
        ],
        out_specs=[
            pl.BlockSpec((bn, C), lambda j: (j, 0)),
            pl.BlockSpec((C, C), lambda j: (0, 0)),
            pl.BlockSpec((8, C), lambda j: (0, 0)),
            pl.BlockSpec((C, CK), lambda j: (0, 0)),
        ],
        out_shape=[
            jax.ShapeDtypeStruct((n, C), jnp.bfloat16),
            jax.ShapeDtypeStruct((C, C), jnp.float32),
            jax.ShapeDtypeStruct((8, C), jnp.float32),
            jax.ShapeDtypeStruct((C, CK), jnp.float32),
        ],
        compiler_params=pltpu.CompilerParams(
            dimension_semantics=("arbitrary",),
            vmem_limit_bytes=56 * 1024 * 1024,
        ),
        name="lga_reduce",
        interpret=interpret,
    )(x, wqT, wkT, wvT, woT)

    out = pl.pallas_call(
        _map_body,
        grid=(p,),
        in_specs=[
            pl.BlockSpec((bm, C), lambda j: (j, 0)),
            pl.BlockSpec((C, CK), lambda j: (0, 0)),
            pl.BlockSpec((1, C), lambda j: (0, 0)),
        ],
        out_specs=pl.BlockSpec((bm, C), lambda j: (j, 0)),
        out_shape=jax.ShapeDtypeStruct((n, C), jnp.float32),
        compiler_params=pltpu.CompilerParams(
            dimension_semantics=("parallel",),
            vmem_limit_bytes=56 * 1024 * 1024,
        ),
        name="lga_map",
        interpret=interpret,
    )(q16, bmat, bo2)

    return out


# speedup vs baseline: 1.2818x; 1.1739x over previous
import functools

import jax
import jax.numpy as jnp
from jax.experimental import pallas as pl
from jax.experimental.pallas import tpu as pltpu

C = 256
CK = C + 128


def _elu1(y):
    return jnp.where(y > 0, y + 1.0, jnp.exp(y))


def _reduce_body(
    x_ref, wqT_ref, wkT_ref, wvT_ref, woT_ref, q_ref, acc_ref, ks_ref, b_ref, *, steps
):
    j = pl.program_id(0)
    xb = x_ref[...].astype(jnp.bfloat16)
    q = _elu1(jnp.dot(xb, wqT_ref[...], preferred_element_type=jnp.float32))
    q_ref[...] = q.astype(jnp.float8_e4m3fn)
    k = _elu1(jnp.dot(xb, wkT_ref[...], preferred_element_type=jnp.float32))
    contrib = jax.lax.dot_general(
        k.astype(jnp.bfloat16),
        xb,
        (((0,), (0,)), ((), ())),
        preferred_element_type=jnp.float32,
    )
    ks_part = jnp.sum(k.reshape(-1, 8, C), axis=0)

    @pl.when(j == 0)
    def _():
        acc_ref[...] = contrib
        ks_ref[...] = ks_part

    @pl.when(j != 0)
    def _():
        acc_ref[...] += contrib
        ks_ref[...] += ks_part

    @pl.when(j == steps - 1)
    def _():
        w2 = jnp.dot(wvT_ref[...], woT_ref[...], preferred_element_type=jnp.float32)
        m = jnp.dot(
            acc_ref[...].astype(jnp.bfloat16),
            w2.astype(jnp.bfloat16),
            preferred_element_type=jnp.float32,
        )
        kcol = jax.lax.dot_general(
            ks_ref[...],
            jnp.ones((8, 128), jnp.float32),
            (((0,), (0,)), ((), ())),
            preferred_element_type=jnp.float32,
        )
        b_ref[...] = jnp.concatenate([m, kcol], axis=1)


def _map_body(q_ref, b_ref, bo_ref, o_ref):
    q16 = q_ref[...].astype(jnp.bfloat16)
    g = jnp.dot(q16, b_ref[...].astype(jnp.bfloat16), preferred_element_type=jnp.float32)
    z = 1.0 / jnp.maximum(g[:, C:], 1e-6)
    z256 = jnp.concatenate([z, z], axis=1)
    o_ref[...] = g[:, :C] * z256 + bo_ref[...]


@functools.partial(jax.jit, static_argnames=("interpret",))
def kernel(x, Wq, Wk, Wv, Wo, bo, interpret=False):
    n = x.shape[0]
    bn = 8192
    bm = 8192
    s = n // bn
    p = n // bm

    wqT = Wq.T.astype(jnp.bfloat16)
    wkT = Wk.T.astype(jnp.bfloat16)
    wvT = Wv.T.astype(jnp.bfloat16)
    woT = Wo.T.astype(jnp.bfloat16)
    bo2 = bo.reshape(1, C)

    q8, _, _, bmat = pl.pallas_call(
        functools.partial(_reduce_body, steps=s),
        grid=(s,),
        in_specs=[
            pl.BlockSpec((bn, C), lambda j: (j, 0)),
            pl.BlockSpec((C, C), lambda j: (0, 0)),
            pl.BlockSpec((C, C), lambda j: (0, 0)),
            pl.BlockSpec((C, C), lambda j: (0, 0)),
            pl.BlockSpec((C, C), lambda j: (0, 0)),
        ],
        out_specs=[
            pl.BlockSpec((bn, C), lambda j: (j, 0)),
            pl.BlockSpec((C, C), lambda j: (0, 0)),
            pl.BlockSpec((8, C), lambda j: (0, 0)),
            pl.BlockSpec((C, CK), lambda j: (0, 0)),
        ],
        out_shape=[
            jax.ShapeDtypeStruct((n, C), jnp.float8_e4m3fn),
            jax.ShapeDtypeStruct((C, C), jnp.float32),
            jax.ShapeDtypeStruct((8, C), jnp.float32),
            jax.ShapeDtypeStruct((C, CK), jnp.float32),
        ],
        compiler_params=pltpu.CompilerParams(
            dimension_semantics=("arbitrary",),
            vmem_limit_bytes=56 * 1024 * 1024,
        ),
        name="lga_reduce",
        interpret=interpret,
    )(x, wqT, wkT, wvT, woT)

    out = pl.pallas_call(
        _map_body,
        grid=(p,),
        in_specs=[
            pl.BlockSpec((bm, C), lambda j: (j, 0)),
            pl.BlockSpec((C, CK), lambda j: (0, 0)),
            pl.BlockSpec((1, C), lambda j: (0, 0)),
        ],
        out_specs=pl.BlockSpec((bm, C), lambda j: (j, 0)),
        out_shape=jax.ShapeDtypeStruct((n, C), jnp.float32),
        compiler_params=pltpu.CompilerParams(
            dimension_semantics=("parallel",),
            vmem_limit_bytes=56 * 1024 * 1024,
        ),
        name="lga_map",
        interpret=interpret,
    )(q8, bmat, bo2)

    return out


# speedup vs baseline: 1.3251x; 1.0338x over previous
import functools

import jax
import jax.numpy as jnp
from jax.experimental import pallas as pl
from jax.experimental.pallas import tpu as pltpu

C = 256
CK = C + 128


def _elu1(y):
    return jnp.where(y > 0, y + 1.0, jnp.exp(y))


def _reduce_body(
    x_ref, wqT_ref, wkT_ref, wvT_ref, woT_ref, q_ref, acc_ref, ks_ref, b_ref, *, steps
):
    j = pl.program_id(0)
    xb = x_ref[...].astype(jnp.bfloat16)
    q = _elu1(jnp.dot(xb, wqT_ref[...], preferred_element_type=jnp.float32))
    q_ref[...] = q.astype(jnp.float8_e4m3fn)
    k = _elu1(jnp.dot(xb, wkT_ref[...], preferred_element_type=jnp.float32))
    contrib = jax.lax.dot_general(
        k.astype(jnp.bfloat16),
        xb,
        (((0,), (0,)), ((), ())),
        preferred_element_type=jnp.float32,
    )
    ks_part = jnp.sum(k.reshape(-1, 8, C), axis=0)

    @pl.when(j == 0)
    def _():
        acc_ref[...] = contrib
        ks_ref[...] = ks_part

    @pl.when(j != 0)
    def _():
        acc_ref[...] += contrib
        ks_ref[...] += ks_part

    @pl.when(j == steps - 1)
    def _():
        w2 = jnp.dot(wvT_ref[...], woT_ref[...], preferred_element_type=jnp.float32)
        m = jnp.dot(
            acc_ref[...].astype(jnp.bfloat16),
            w2.astype(jnp.bfloat16),
            preferred_element_type=jnp.float32,
        )
        kcol = jax.lax.dot_general(
            ks_ref[...],
            jnp.ones((8, 128), jnp.float32),
            (((0,), (0,)), ((), ())),
            preferred_element_type=jnp.float32,
        )
        b_ref[...] = jnp.concatenate([m, kcol], axis=1)


def _map_body(q_ref, b_ref, bo_ref, o_ref):
    q16 = q_ref[...].astype(jnp.bfloat16)
    g = jnp.dot(q16, b_ref[...].astype(jnp.bfloat16), preferred_element_type=jnp.float32)
    z = 1.0 / jnp.maximum(g[:, C:], 1e-6)
    z256 = jnp.concatenate([z, z], axis=1)
    o_ref[...] = g[:, :C] * z256 + bo_ref[...]


@functools.partial(jax.jit, static_argnames=("interpret",))
def kernel(x, Wq, Wk, Wv, Wo, bo, interpret=False):
    n = x.shape[0]
    bn = 16384
    bm = 16384
    s = n // bn
    p = n // bm

    wqT = Wq.T.astype(jnp.bfloat16)
    wkT = Wk.T.astype(jnp.bfloat16)
    wvT = Wv.T.astype(jnp.bfloat16)
    woT = Wo.T.astype(jnp.bfloat16)
    bo2 = bo.reshape(1, C)

    q8, _, _, bmat = pl.pallas_call(
        functools.partial(_reduce_body, steps=s),
        grid=(s,),
        in_specs=[
            pl.BlockSpec((bn, C), lambda j: (j, 0)),
            pl.BlockSpec((C, C), lambda j: (0, 0)),
            pl.BlockSpec((C, C), lambda j: (0, 0)),
            pl.BlockSpec((C, C), lambda j: (0, 0)),
            pl.BlockSpec((C, C), lambda j: (0, 0)),
        ],
        out_specs=[
            pl.BlockSpec((bn, C), lambda j: (j, 0)),
            pl.BlockSpec((C, C), lambda j: (0, 0)),
            pl.BlockSpec((8, C), lambda j: (0, 0)),
            pl.BlockSpec((C, CK), lambda j: (0, 0)),
        ],
        out_shape=[
            jax.ShapeDtypeStruct((n, C), jnp.float8_e4m3fn),
            jax.ShapeDtypeStruct((C, C), jnp.float32),
            jax.ShapeDtypeStruct((8, C), jnp.float32),
            jax.ShapeDtypeStruct((C, CK), jnp.float32),
        ],
        compiler_params=pltpu.CompilerParams(
            dimension_semantics=("arbitrary",),
            vmem_limit_bytes=56 * 1024 * 1024,
        ),
        name="lga_reduce",
        interpret=interpret,
    )(x, wqT, wkT, wvT, woT)

    out = pl.pallas_call(
        _map_body,
        grid=(p,),
        in_specs=[
            pl.BlockSpec((bm, C), lambda j: (j, 0)),
            pl.BlockSpec((C, CK), lambda j: (0, 0)),
            pl.BlockSpec((1, C), lambda j: (0, 0)),
        ],
        out_specs=pl.BlockSpec((bm, C), lambda j: (j, 0)),
        out_shape=jax.ShapeDtypeStruct((n, C), jnp.float32),
        compiler_params=pltpu.CompilerParams(
            dimension_semantics=("parallel",),
            vmem_limit_bytes=56 * 1024 * 1024,
        ),
        name="lga_map",
        interpret=interpret,
    )(q8, bmat, bo2)

    return out


# speedup vs baseline: 1.3254x; 1.0002x over previous
import functools

import jax
import jax.numpy as jnp
from jax.experimental import pallas as pl
from jax.experimental.pallas import tpu as pltpu

C = 256
CK = C + 128


def _elu1(y):
    return jnp.where(y > 0, y + 1.0, jnp.exp(y))


def _reduce_body(
    x_ref, wqT_ref, wkT_ref, wvT_ref, woT_ref, q_ref, acc_ref, ks_ref, b_ref, *, steps
):
    j = pl.program_id(0)
    xb = x_ref[...].astype(jnp.bfloat16)
    q = _elu1(jnp.dot(xb, wqT_ref[...], preferred_element_type=jnp.float32))
    q_ref[...] = q.astype(jnp.float8_e4m3fn)
    k = _elu1(jnp.dot(xb, wkT_ref[...], preferred_element_type=jnp.float32))
    contrib = jax.lax.dot_general(
        k.astype(jnp.bfloat16),
        xb,
        (((0,), (0,)), ((), ())),
        preferred_element_type=jnp.float32,
    )
    ks_part = jnp.sum(k.reshape(-1, 8, C), axis=0)

    @pl.when(j == 0)
    def _():
        acc_ref[...] = contrib
        ks_ref[...] = ks_part

    @pl.when(j != 0)
    def _():
        acc_ref[...] += contrib
        ks_ref[...] += ks_part

    @pl.when(j == steps - 1)
    def _():
        w2 = jnp.dot(wvT_ref[...], woT_ref[...], preferred_element_type=jnp.float32)
        m = jnp.dot(
            acc_ref[...].astype(jnp.bfloat16),
            w2.astype(jnp.bfloat16),
            preferred_element_type=jnp.float32,
        )
        kcol = jax.lax.dot_general(
            ks_ref[...],
            jnp.ones((8, 128), jnp.float32),
            (((0,), (0,)), ((), ())),
            preferred_element_type=jnp.float32,
        )
        b_ref[...] = jnp.concatenate([m, kcol], axis=1)


def _map_body(q_ref, b_ref, bo_ref, o_ref):
    q16 = q_ref[...].astype(jnp.bfloat16)
    g = jnp.dot(q16, b_ref[...].astype(jnp.bfloat16), preferred_element_type=jnp.float32)
    z = 1.0 / jnp.maximum(g[:, C:], 1e-6)
    z256 = jnp.concatenate([z, z], axis=1)
    o_ref[...] = g[:, :C] * z256 + bo_ref[...]


@jax.jit
def kernel(x, Wq, Wk, Wv, Wo, bo):
    n = x.shape[0]
    bn = 16384
    bm = 16384
    s = n // bn
    p = n // bm

    wqT = Wq.T.astype(jnp.bfloat16)
    wkT = Wk.T.astype(jnp.bfloat16)
    wvT = Wv.T.astype(jnp.bfloat16)
    woT = Wo.T.astype(jnp.bfloat16)
    bo2 = bo.reshape(1, C)

    q8, _, _, bmat = pl.pallas_call(
        functools.partial(_reduce_body, steps=s),
        grid=(s,),
        in_specs=[
            pl.BlockSpec((bn, C), lambda j: (j, 0)),
            pl.BlockSpec((C, C), lambda j: (0, 0)),
            pl.BlockSpec((C, C), lambda j: (0, 0)),
            pl.BlockSpec((C, C), lambda j: (0, 0)),
            pl.BlockSpec((C, C), lambda j: (0, 0)),
        ],
        out_specs=[
            pl.BlockSpec((bn, C), lambda j: (j, 0)),
            pl.BlockSpec((C, C), lambda j: (0, 0)),
            pl.BlockSpec((8, C), lambda j: (0, 0)),
            pl.BlockSpec((C, CK), lambda j: (0, 0)),
        ],
        out_shape=[
            jax.ShapeDtypeStruct((n, C), jnp.float8_e4m3fn),
            jax.ShapeDtypeStruct((C, C), jnp.float32),
            jax.ShapeDtypeStruct((8, C), jnp.float32),
            jax.ShapeDtypeStruct((C, CK), jnp.float32),
        ],
        compiler_params=pltpu.CompilerParams(
            dimension_semantics=("arbitrary",),
            vmem_limit_bytes=56 * 1024 * 1024,
        ),
        name="lga_reduce",
    )(x, wqT, wkT, wvT, woT)

    out = pl.pallas_call(
        _map_body,
        grid=(p,),
        in_specs=[
            pl.BlockSpec((bm, C), lambda j: (j, 0)),
            pl.BlockSpec((C, CK), lambda j: (0, 0)),
            pl.BlockSpec((1, C), lambda j: (0, 0)),
        ],
        out_specs=pl.BlockSpec((bm, C), lambda j: (j, 0)),
        out_shape=jax.ShapeDtypeStruct((n, C), jnp.float32),
        compiler_params=pltpu.CompilerParams(
            dimension_semantics=("parallel",),
            vmem_limit_bytes=56 * 1024 * 1024,
        ),
        name="lga_map",
    )(q8, bmat, bo2)

    return out
